# baseline (speedup 1.0000x reference)
import numpy as np
import jax
import jax.numpy as jnp
from jax.experimental import pallas as pl

B, N, H, E = 16, 325, 64, 5200
TAU = 0.1
G3 = 3 * H
EP = 5376
TE = 896
NCHUNK = EP // TE
KB = 4
NP = 328
BF = jnp.bfloat16
F32 = jnp.float32


def _sp(a):
    ah = a.astype(BF)
    al = (a - ah.astype(F32)).astype(BF)
    return ah, al


def _dgb(a, b, ca, cb):
    return jax.lax.dot_general(
        a, b, (((ca,), (cb,)), ((), ())), preferred_element_type=F32)


def _dot3(asp, bsp, ca=1, cb=0):
    ah, al = asp
    bh, bl = bsp
    return (_dgb(ah, bh, ca, cb) + _dgb(ah, bl, ca, cb)
            + _dgb(al, bh, ca, cb))


def _dot2(mask_bf, bsp, ca, cb):
    bh, bl = bsp
    return _dgb(mask_bf, bh, ca, cb) + _dgb(mask_bf, bl, ca, cb)


def _gelu(x):
    return 0.5 * x * (1.0 + jax.lax.erf(x * np.float32(1.0 / np.sqrt(2.0))))


def _comb(gi, gh, hprev):
    r = jax.nn.sigmoid(gi[:, :H] + gh[:, :H])
    z = jax.nn.sigmoid(gi[:, H:2 * H] + gh[:, H:2 * H])
    n = jnp.tanh(gi[:, 2 * H:] + r * gh[:, 2 * H:])
    out = (1.0 - z) * n
    if hprev is not None:
        out = out + z * hprev
    return out


def _combP(po, hprev):
    r = jax.nn.sigmoid(po[:, :H])
    z = jax.nn.sigmoid(po[:, H:2 * H])
    n = jnp.tanh(po[:, 2 * H:3 * H] + r * po[:, 3 * H:])
    return (1.0 - z) * n + z * hprev


def _body(h_ref, dst_ref, src_ref, g_ref,
          wn_ref, bhf0_ref, bhr0_ref,
          wa1_ref, ba1_ref, wa2_ref, ba2_ref,
          wb1_ref, bb1_ref, wb2_ref, bb2_ref,
          wc1_ref, bc1_ref,
          bhr1_ref, bhf1_ref,
          wc_ref, bc_ref,
          out_ref):
    hall = h_ref[...].reshape(KB * NP, H)
    dst = dst_ref[...]
    src = src_ref[...]
    g = g_ref[...].reshape(KB, EP)

    wsp = lambda ref: (ref[0], ref[1])

    hsp = _sp(hall)
    gnode = _dot3(hsp, wsp(wn_ref))
    gf = gnode[:, :G3] + bhf0_ref[0:1]
    gr = gnode[:, G3:] + bhr0_ref[0:1]
    hf1 = _comb(gf, jnp.broadcast_to(bhf0_ref[1:2], (KB * NP, G3)), None)
    hr2 = _comb(gr, jnp.broadcast_to(bhr0_ref[1:2], (KB * NP, G3)), None)

    td = _sp(jnp.concatenate(
        [jnp.concatenate([hall[b * NP:(b + 1) * NP], hf1[b * NP:(b + 1) * NP]],
                         axis=1) for b in range(KB)], axis=1))
    ts = _sp(jnp.concatenate(
        [jnp.concatenate([hr2[b * NP:(b + 1) * NP], hall[b * NP:(b + 1) * NP]],
                         axis=1) for b in range(KB)], axis=1))

    logit_parts = []
    hs_parts = []
    sd_masks = []
    R = KB * TE
    for c in range(NCHUNK):
        sl = slice(c * TE, (c + 1) * TE)
        dst_c = dst[:, sl]
        src_c = src[:, sl]
        iota_c = jax.lax.broadcasted_iota(jnp.int32, (NP, TE), 0)
        sd_c = (iota_c == dst_c).astype(BF)
        ss_c = (iota_c == src_c).astype(BF)
        sd_masks.append(sd_c)
        gd = _dot2(sd_c, td, 0, 0)
        gs = _dot2(ss_c, ts, 0, 0)
        hd = jnp.concatenate([gd[:, b * 2 * H:b * 2 * H + H]
                              for b in range(KB)], axis=0)
        f1d = jnp.concatenate([gd[:, b * 2 * H + H:(b + 1) * 2 * H]
                               for b in range(KB)], axis=0)
        r2s = jnp.concatenate([gs[:, b * 2 * H:b * 2 * H + H]
                               for b in range(KB)], axis=0)
        hs = jnp.concatenate([gs[:, b * 2 * H + H:(b + 1) * 2 * H]
                              for b in range(KB)], axis=0)

        po1 = _dot3(_sp(jnp.concatenate([hd, r2s], axis=1)),
                    wsp(wa1_ref)) + ba1_ref[...]
        hr1 = _combP(po1, r2s)
        po2 = _dot3(_sp(jnp.concatenate([hs, f1d], axis=1)),
                    wsp(wa2_ref)) + ba2_ref[...]
        hf2 = _combP(po2, f1d)

        gi1 = _dot3(_sp(jnp.concatenate([hf2, r2s], axis=1)),
                    wsp(wb1_ref)) + bb1_ref[...]
        hr2l1 = _comb(gi1, jnp.broadcast_to(bhr1_ref[...], (R, G3)), None)
        gif = _dot3(_sp(jnp.concatenate([f1d, hr1], axis=1)),
                    wsp(wb2_ref)) + bb2_ref[...]
        m7 = _dot3(_sp(hr2l1), wsp(wc1_ref)) + bc1_ref[...]
        o_r = _comb(gif[:, :G3], m7, hr2l1)
        o_f = _comb(gif[:, G3:], jnp.broadcast_to(bhf1_ref[...], (R, G3)),
                    None)

        x = (_dot3(wsp(wc_ref),
                   _sp(jnp.concatenate([o_f, o_r], axis=1)), 1, 1)
             + bc_ref[0, 0])
        logit_parts.append(_gelu(x).reshape(KB, TE))
        hs_parts.append(hs)

    logit = jnp.concatenate(logit_parts, axis=1)
    lanes = jax.lax.broadcasted_iota(jnp.int32, (1, EP), 1)
    valid = lanes < E
    logit = jnp.where(valid, logit, np.float32(-1e30))

    iota_full = jax.lax.broadcasted_iota(jnp.int32, (NP, EP), 0)
    bs = iota_full == src

    def _seg_sum(row):
        return jnp.sum(jnp.where(bs, row, 0.0), axis=1, keepdims=True)

    def _seg_gather(col):
        return jnp.sum(jnp.where(bs, jnp.broadcast_to(col, (NP, EP)), 0.0),
                       axis=0, keepdims=True)

    zh = (logit + g) * np.float32(1.0 / TAU)
    es_all = jnp.where(valid, jnp.exp(logit), 0.0)
    coef_rows = []
    for b in range(KB):
        zb = zh[b:b + 1]
        m = jnp.max(jnp.where(bs, zb, np.float32(-1e30)), axis=1,
                    keepdims=True)
        eh = jnp.exp(zb - _seg_gather(m))
        eh = jnp.where(valid, eh, 0.0)
        es = es_all[b:b + 1]
        den = _seg_gather(_seg_sum(eh) * _seg_sum(es))
        coef_rows.append(jnp.where(valid,
                                   eh * es / (den + np.float32(1e-12)), 0.0))

    acc = jnp.zeros((NP, KB * H), F32)
    for c in range(NCHUNK):
        sl = slice(c * TE, (c + 1) * TE)
        msg = jnp.concatenate(
            [hs_parts[c][b * TE:(b + 1) * TE]
             * coef_rows[b][:, sl].reshape(TE, 1) for b in range(KB)],
            axis=1)
        acc = acc + _dot2(sd_masks[c], _sp(msg), 1, 0)
    for b in range(KB):
        out_ref[b] = acc[:, b * H:(b + 1) * H]


def _splitw(w):
    wh = w.astype(BF)
    wl = (w - wh.astype(F32)).astype(BF)
    return jnp.stack([wh, wl])


def kernel(h, params, edge_index):
    f32 = jnp.float32
    p0, p1 = params['l0'], params['l1']
    a1 = p1['W_ih_r'][:, :H]
    a2 = p1['W_ih_r'][:, H:]
    f1 = p1['W_ih_f'][:, :H]
    f2 = p1['W_ih_f'][:, H:]

    wn = _splitw(jnp.concatenate([p0['W_ih_f'].T, p0['W_ih_r'].T], axis=1))
    bhf0 = jnp.stack([p0['b_ih_f'], p0['b_hh_f']], axis=0)
    bhr0 = jnp.stack([p0['b_ih_r'], p0['b_hh_r']], axis=0)

    zH = jnp.zeros((H, H), f32)

    def _packA(wih, whh, bih, bhh):
        top = jnp.concatenate([wih.T[:, :2 * H], wih.T[:, 2 * H:], zH],
                              axis=1)
        bot = jnp.concatenate([whh.T[:, :2 * H], zH, whh.T[:, 2 * H:]],
                              axis=1)
        w = jnp.concatenate([top, bot], axis=0)
        b = jnp.concatenate([(bih + bhh)[:2 * H], bih[2 * H:], bhh[2 * H:]])
        return _splitw(w), b[None, :]

    wa1, ba1 = _packA(p0['W_ih_r'], p0['W_hh_r'],
                      p0['b_ih_r'], p0['b_hh_r'])
    wa2, ba2 = _packA(p0['W_ih_f'], p0['W_hh_f'],
                      p0['b_ih_f'], p0['b_hh_f'])
    wb1 = _splitw(jnp.concatenate([a1.T, a2.T], axis=0))
    bb1 = p1['b_ih_r'][None, :]
    wb2 = _splitw(jnp.concatenate(
        [jnp.concatenate([a1.T, a2.T], axis=0),
         jnp.concatenate([f1.T, f2.T], axis=0)], axis=1))
    bb2 = jnp.concatenate([p1['b_ih_r'], p1['b_ih_f']])[None, :]
    wc1 = _splitw(p1['W_hh_r'].T)
    bc1 = p1['b_hh_r'][None, :]
    bhr1 = p1['b_hh_r'][None, :]
    bhf1 = p1['b_hh_f'][None, :]
    wc = _splitw(params['Wc'][0:1, :])
    bc = params['bc'][None, :]

    u = jax.random.uniform(jax.random.key(42), (E, B),
                           minval=1e-6, maxval=1.0 - 1e-6)
    g = -jnp.log(-jnp.log(u))
    gp = jnp.zeros((B, 1, EP), f32).at[:, 0, :E].set(jnp.transpose(g))

    pad = jnp.full((1, EP - E), N, jnp.int32)
    dstr = jnp.concatenate([edge_index[0][None, :], pad], axis=1)
    srcr = jnp.concatenate([edge_index[1][None, :], pad], axis=1)

    hp = jnp.zeros((B, NP, H), f32).at[:, :N, :].set(h)

    full = lambda shape: pl.BlockSpec(shape, lambda b: (0,) * len(shape))
    grid_spec = pl.GridSpec(
        grid=(B // KB,),
        in_specs=[
            pl.BlockSpec((KB, NP, H), lambda b: (b, 0, 0)),
            full((1, EP)), full((1, EP)),
            pl.BlockSpec((KB, 1, EP), lambda b: (b, 0, 0)),
            full((2, H, 2 * G3)), full((2, G3)), full((2, G3)),
            full((2, 2 * H, 4 * H)), full((1, 4 * H)),
            full((2, 2 * H, 4 * H)), full((1, 4 * H)),
            full((2, 2 * H, G3)), full((1, G3)),
            full((2, 2 * H, 2 * G3)), full((1, 2 * G3)),
            full((2, H, G3)), full((1, G3)),
            full((1, G3)), full((1, G3)),
            full((2, 1, 2 * H)), full((1, 1)),
        ],
        out_specs=pl.BlockSpec((KB, NP, H), lambda b: (b, 0, 0)),
    )
    out = pl.pallas_call(
        _body,
        grid_spec=grid_spec,
        out_shape=jax.ShapeDtypeStruct((B, NP, H), f32),
    )(hp, dstr, srcr, gp,
      wn, bhf0, bhr0,
      wa1, ba1, wa2, ba2,
      wb1, bb1, wb2, bb2,
      wc1, bc1,
      bhr1, bhf1,
      wc, bc)
    return out[:, :N, :]
